# baseline (speedup 1.0000x reference)
import functools

import jax
import jax.numpy as jnp
from jax import lax
from jax.experimental import pallas as pl
from jax.experimental.pallas import tpu as pltpu
from jax.experimental.pallas import tpu_sc as plsc

N = 100000
E = 3200000
RN, CN = 784, 128
NP = RN * CN
NC, NS = 2, 16
NW = NC * NS
EPW = 102400
EP = EPW * NW
SB = 6400
SB6 = 3200
NPS = NP // NS

_MESH = plsc.VectorSubcoreMesh(core_axis_name="c", subcore_axis_name="s")


def _lrelu(h):
    return jnp.where(h >= 0, h, 0.1 * h)


def _deg_body(dst_hbm, ones_hbm, zeros_hbm, out_hbm,
              dstb0, dstb1, ones_v, acc_sh, lsem0, lsem1):
    cid = lax.axis_index("c")
    sid = lax.axis_index("s")
    pltpu.sync_copy(zeros_hbm.at[pl.ds(sid * NPS, NPS)],
                    acc_sh.at[pl.ds(sid * NPS, NPS)])
    pltpu.sync_copy(ones_hbm, ones_v)
    plsc.subcore_barrier()
    ebase = (cid * NS + sid) * EPW
    NSC = EPW // SB
    dstb = (dstb0, dstb1)
    lsem = (lsem0, lsem1)

    def lin(s, b, sem):
        return pltpu.async_copy(dst_hbm.at[pl.ds(ebase + s * SB, SB)],
                                dstb[b], sem)

    lin(0, 0, lsem[0])

    def loop(tt, c):
        s0 = 2 * tt
        lin(s0 + 1, 1, lsem[1])
        pltpu.make_async_copy(dst_hbm.at[pl.ds(ebase, SB)],
                              dstb[0], lsem[0]).wait()
        pltpu.sync_copy(ones_v, acc_sh.at[dstb[0]], add=True)
        lin(jnp.minimum(s0 + 2, NSC - 1), 0, lsem[0])
        pltpu.make_async_copy(dst_hbm.at[pl.ds(ebase, SB)],
                              dstb[1], lsem[1]).wait()
        pltpu.sync_copy(ones_v, acc_sh.at[dstb[1]], add=True)
        return c

    lax.fori_loop(0, NSC // 2, loop, 0)
    pltpu.make_async_copy(dst_hbm.at[pl.ds(ebase, SB)], dstb[0], lsem[0]).wait()
    plsc.subcore_barrier()
    pltpu.sync_copy(acc_sh.at[pl.ds(sid * NPS, NPS)],
                    out_hbm.at[cid, pl.ds(sid * NPS, NPS)])


_deg_sc = functools.partial(
    pl.kernel,
    out_type=jax.ShapeDtypeStruct((NC, NP), jnp.float32),
    mesh=_MESH,
    scratch_types=[
        pltpu.VMEM((SB,), jnp.int32),
        pltpu.VMEM((SB,), jnp.int32),
        pltpu.VMEM((SB,), jnp.float32),
        pltpu.VMEM_SHARED((NP,), jnp.float32),
        pltpu.SemaphoreType.DMA,
        pltpu.SemaphoreType.DMA,
    ],
)(_deg_body)


def _make_prop(K, SB):
    def body(*refs):
        src_hbm, dst_hbm = refs[0], refs[1]
        us = refs[2:2 + K]
        zeros_hbm = refs[2 + K]
        outs = refs[3 + K:3 + 2 * K]
        r = 3 + 2 * K
        srcb = refs[r:r + 2]
        dstb = refs[r + 2:r + 4]
        vals = (refs[r + 4:r + 4 + K], refs[r + 4 + K:r + 4 + 2 * K])
        accs = refs[r + 4 + 2 * K:r + 4 + 3 * K]
        lsem = refs[r + 4 + 3 * K:r + 6 + 3 * K]
        gsem = refs[r + 6 + 3 * K:r + 8 + 3 * K]

        cid = lax.axis_index("c")
        sid = lax.axis_index("s")
        for k in range(K):
            pltpu.sync_copy(zeros_hbm.at[pl.ds(sid * NPS, NPS)],
                            accs[k].at[pl.ds(sid * NPS, NPS)])
        plsc.subcore_barrier()
        ebase = (cid * NS + sid) * EPW
        NSC = EPW // SB

        def lin(s, b):
            off = ebase + s * SB
            pltpu.async_copy(src_hbm.at[pl.ds(off, SB)], srcb[b], lsem[b])
            pltpu.async_copy(dst_hbm.at[pl.ds(off, SB)], dstb[b], lsem[b])

        def wait_lin(b):
            pltpu.make_async_copy(src_hbm.at[pl.ds(ebase, SB)],
                                  srcb[b], lsem[b]).wait()
            pltpu.make_async_copy(src_hbm.at[pl.ds(ebase, SB)],
                                  dstb[b], lsem[b]).wait()

        def gather(b):
            for k in range(K):
                pltpu.async_copy(us[k].at[srcb[b]], vals[b][k], gsem[b])

        def wait_gather(b):
            for k in range(K):
                pltpu.make_async_copy(us[k].at[srcb[b]],
                                      vals[b][k], gsem[b]).wait()

        def scatter(b):
            for k in range(K):
                pltpu.sync_copy(vals[b][k], accs[k].at[dstb[b]], add=True)

        lin(0, 0)

        def loop(tt, c):
            s0 = 2 * tt
            lin(s0 + 1, 1)
            wait_lin(0)
            gather(0)
            wait_lin(1)
            gather(1)
            wait_gather(0)
            scatter(0)
            lin(jnp.minimum(s0 + 2, NSC - 1), 0)
            wait_gather(1)
            scatter(1)
            return c

        lax.fori_loop(0, NSC // 2, loop, 0)
        wait_lin(0)
        plsc.subcore_barrier()
        for k in range(K):
            pltpu.sync_copy(accs[k].at[pl.ds(sid * NPS, NPS)],
                            outs[k].at[cid, pl.ds(sid * NPS, NPS)])

    return functools.partial(
        pl.kernel,
        out_type=[jax.ShapeDtypeStruct((NC, NP), jnp.float32)] * K,
        mesh=_MESH,
        scratch_types=(
            [pltpu.VMEM((SB,), jnp.int32)] * 4
            + [pltpu.VMEM((SB,), jnp.float32)] * (2 * K)
            + [pltpu.VMEM_SHARED((NP,), jnp.float32)] * K
            + [pltpu.SemaphoreType.DMA] * 4
        ),
    )(body)


_prop1 = _make_prop(1, SB)
_prop6 = _make_prop(6, SB6)


def _tc_dinv_body(d0, d1, x0, x1, x2, x3, x4, x5, dinv_o, u0, u1, u2, u3, u4, u5):
    dinv = lax.rsqrt(d0[...] + d1[...] + 1.0)
    dinv_o[...] = dinv
    for xr, ur in ((x0, u0), (x1, u1), (x2, u2), (x3, u3), (x4, u4), (x5, u5)):
        ur[...] = dinv * xr[...]


_tc_dinv = pl.pallas_call(
    _tc_dinv_body,
    out_shape=[jax.ShapeDtypeStruct((RN, CN), jnp.float32)] * 7,
)


def _tc_l1_body(*refs):
    dinv_r = refs[0]
    a0 = refs[1:7]
    a1 = refs[7:13]
    u0 = refs[13:19]
    w11, b11, w12 = refs[19], refs[20], refs[21]
    out_u = refs[22]
    dinv = dinv_r[...]
    q = [dinv * (a0[j][...] + a1[j][...] + u0[j][...]) for j in range(6)]
    t = jnp.zeros((RN, CN), jnp.float32)
    for j in range(16):
        h = b11[j]
        for i in range(6):
            h = h + q[i] * w11[i * 16 + j]
        t = t + _lrelu(h) * w12[j]
    out_u[...] = dinv * t


_tc_l1 = pl.pallas_call(
    _tc_l1_body,
    out_shape=jax.ShapeDtypeStruct((RN, CN), jnp.float32),
)


def _tc_ymap_body(dinv_r, a0, a1, u, b, y_o, un_o):
    dinv = dinv_r[...]
    y = _lrelu(dinv * (a0[...] + a1[...] + u[...]) + b[0])
    y_o[...] = y
    un_o[...] = dinv * y


_tc_ymap = pl.pallas_call(
    _tc_ymap_body,
    out_shape=[jax.ShapeDtypeStruct((RN, CN), jnp.float32)] * 2,
)


def _tc_tmap_body(dinv_r, a0, a1, u, wa, ba, wb, un_o):
    dinv = dinv_r[...]
    z = dinv * (a0[...] + a1[...] + u[...])
    t = jnp.zeros((RN, CN), jnp.float32)
    for j in range(16):
        t = t + _lrelu(z * wa[j] + ba[j]) * wb[j]
    un_o[...] = dinv * t


_tc_tmap = pl.pallas_call(
    _tc_tmap_body,
    out_shape=jax.ShapeDtypeStruct((RN, CN), jnp.float32),
)


def kernel(x, edge_index, W11, b11, W11r, b11r, W12, b12, W21, b21, W22, b22,
           W31, b31, W32, b32):
    ei = edge_index.astype(jnp.int32)
    pad = jnp.full((EP - E,), N, jnp.int32)
    src_p = jnp.concatenate([ei[0], pad])
    dst_p = jnp.concatenate([ei[1], pad])
    zeros = jnp.zeros((NP,), jnp.float32)
    ones_kr = jnp.ones((SB,), jnp.float32)
    xp = jnp.pad(x, ((0, NP - N), (0, 0)))

    degp = _deg_sc(dst_p, ones_kr, zeros)
    xcols = [xp[:, j].reshape(RN, CN) for j in range(6)]
    dinv2, *u0 = _tc_dinv(degp[0].reshape(RN, CN), degp[1].reshape(RN, CN),
                          *xcols)

    acc6 = _prop6(src_p, dst_p, *[u.reshape(NP) for u in u0], zeros)
    w11f, w12f = W11.reshape(-1), W12.reshape(-1)
    w11rf = W11r.reshape(-1)
    w21f, w22f = W21.reshape(-1), W22.reshape(-1)
    w31f, w32f = W31.reshape(-1), W32.reshape(-1)
    u = _tc_l1(dinv2,
               *[a[0].reshape(RN, CN) for a in acc6],
               *[a[1].reshape(RN, CN) for a in acc6],
               *u0, w11f, b11, w12f)

    seq = [
        (b12, (w11rf, b11r, w12f)),
        (b12, (w21f, b21, w22f)),
        (b22, (w21f, b21, w22f)),
        (b22, (w31f, b31, w32f)),
        (b32, (w31f, b31, w32f)),
        (b32, None),
    ]
    ys = []
    for bout, nxt in seq:
        acc = _prop1(src_p, dst_p, u.reshape(NP), zeros)[0]
        y, uy = _tc_ymap(dinv2, acc[0].reshape(RN, CN), acc[1].reshape(RN, CN),
                         u, bout)
        ys.append(y)
        if nxt is not None:
            wa, ba, wb = nxt
            acc = _prop1(src_p, dst_p, uy.reshape(NP), zeros)[0]
            u = _tc_tmap(dinv2, acc[0].reshape(RN, CN),
                         acc[1].reshape(RN, CN), uy, wa, ba, wb)

    return tuple(y.reshape(NP)[:N].reshape(N, 1) for y in ys)


# speedup vs baseline: 3.3859x; 3.3859x over previous
import functools

import jax
import jax.numpy as jnp
from jax import lax
from jax.experimental import pallas as pl
from jax.experimental.pallas import tpu as pltpu
from jax.experimental.pallas import tpu_sc as plsc

N = 100000
E = 3200000
RN, CN = 784, 128
NP = RN * CN
NC, NS = 2, 16
NW = NC * NS
EPW = 102400
EP = EPW * NW
SB = 6400
SB6 = 3200
NPS = NP // NS

_MESH = plsc.VectorSubcoreMesh(core_axis_name="c", subcore_axis_name="s")


def _lrelu(h):
    return jnp.where(h >= 0, h, 0.1 * h)


def _deg_body(dst_hbm, ones_hbm, zeros_hbm, out_hbm,
              dstb0, dstb1, ones_v, acc_sh, lsem0, lsem1):
    cid = lax.axis_index("c")
    sid = lax.axis_index("s")
    pltpu.sync_copy(zeros_hbm.at[pl.ds(sid * NPS, NPS)],
                    acc_sh.at[pl.ds(sid * NPS, NPS)])
    pltpu.sync_copy(ones_hbm, ones_v)
    plsc.subcore_barrier()
    ebase = (cid * NS + sid) * EPW
    NSC = EPW // SB
    dstb = (dstb0, dstb1)
    lsem = (lsem0, lsem1)

    def lin(s, b, sem):
        return pltpu.async_copy(dst_hbm.at[pl.ds(ebase + s * SB, SB)],
                                dstb[b], sem)

    lin(0, 0, lsem[0])

    def loop(tt, c):
        s0 = 2 * tt
        lin(s0 + 1, 1, lsem[1])
        pltpu.make_async_copy(dst_hbm.at[pl.ds(ebase, SB)],
                              dstb[0], lsem[0]).wait()
        pltpu.sync_copy(ones_v, acc_sh.at[dstb[0]], add=True)
        lin(jnp.minimum(s0 + 2, NSC - 1), 0, lsem[0])
        pltpu.make_async_copy(dst_hbm.at[pl.ds(ebase, SB)],
                              dstb[1], lsem[1]).wait()
        pltpu.sync_copy(ones_v, acc_sh.at[dstb[1]], add=True)
        return c

    lax.fori_loop(0, NSC // 2, loop, 0)
    pltpu.make_async_copy(dst_hbm.at[pl.ds(ebase, SB)], dstb[0], lsem[0]).wait()
    plsc.subcore_barrier()
    pltpu.sync_copy(acc_sh.at[pl.ds(sid * NPS, NPS)],
                    out_hbm.at[cid, pl.ds(sid * NPS, NPS)])


_deg_sc = functools.partial(
    pl.kernel,
    out_type=jax.ShapeDtypeStruct((NC, NP), jnp.float32),
    mesh=_MESH,
    scratch_types=[
        pltpu.VMEM((SB,), jnp.int32),
        pltpu.VMEM((SB,), jnp.int32),
        pltpu.VMEM((SB,), jnp.float32),
        pltpu.VMEM_SHARED((NP,), jnp.float32),
        pltpu.SemaphoreType.DMA,
        pltpu.SemaphoreType.DMA,
    ],
)(_deg_body)


def _make_prop1(SB):
    NSC = EPW // SB

    def body(src_hbm, dst_hbm, u_hbm, zeros_hbm, out_hbm,
             u_loc, srcb0, srcb1, dstb0, dstb1, vals0, vals1,
             acc_sh, lsem0, lsem1, ssem0, ssem1):
        srcb = (srcb0, srcb1)
        dstb = (dstb0, dstb1)
        vals = (vals0, vals1)
        lsem = (lsem0, lsem1)
        ssem = (ssem0, ssem1)
        cid = lax.axis_index("c")
        sid = lax.axis_index("s")
        pltpu.sync_copy(zeros_hbm.at[pl.ds(sid * NPS, NPS)],
                        acc_sh.at[pl.ds(sid * NPS, NPS)])
        pltpu.sync_copy(u_hbm, u_loc)
        plsc.subcore_barrier()
        ebase = (cid * NS + sid) * EPW

        def lin(s, b):
            off = ebase + s * SB
            pltpu.async_copy(src_hbm.at[pl.ds(off, SB)], srcb[b], lsem[b])
            pltpu.async_copy(dst_hbm.at[pl.ds(off, SB)], dstb[b], lsem[b])

        def wait_lin(b):
            pltpu.make_async_copy(src_hbm.at[pl.ds(ebase, SB)],
                                  srcb[b], lsem[b]).wait()
            pltpu.make_async_copy(src_hbm.at[pl.ds(ebase, SB)],
                                  dstb[b], lsem[b]).wait()

        def gather_compute(b):
            def g(i, c):
                idx = srcb[b][pl.ds(i * 16, 16)]
                vals[b][pl.ds(i * 16, 16)] = plsc.load_gather(u_loc, [idx])
                return c
            lax.fori_loop(0, SB // 16, g, 0)

        def issue_scatter(b):
            pltpu.async_copy(vals[b], acc_sh.at[dstb[b]], ssem[b], add=True)

        def wait_scatter(b):
            pltpu.make_async_copy(vals[b], acc_sh.at[dstb[b]], ssem[b]).wait()

        lin(0, 0)
        lin(1, 1)

        def loop(tt, c):
            s0 = 2 * tt
            wait_lin(0)
            gather_compute(0)
            issue_scatter(0)
            wait_lin(1)
            gather_compute(1)
            wait_scatter(0)
            lin(jnp.minimum(s0 + 2, NSC - 1), 0)
            issue_scatter(1)
            wait_scatter(1)
            lin(jnp.minimum(s0 + 3, NSC - 1), 1)
            return c

        lax.fori_loop(0, NSC // 2, loop, 0)
        wait_lin(0)
        wait_lin(1)
        plsc.subcore_barrier()
        pltpu.sync_copy(acc_sh.at[pl.ds(sid * NPS, NPS)],
                        out_hbm.at[cid, pl.ds(sid * NPS, NPS)])

    return functools.partial(
        pl.kernel,
        out_type=jax.ShapeDtypeStruct((NC, NP), jnp.float32),
        mesh=_MESH,
        compiler_params=pltpu.CompilerParams(needs_layout_passes=False),
        scratch_types=(
            [pltpu.VMEM((NP,), jnp.float32)]
            + [pltpu.VMEM((SB,), jnp.int32)] * 4
            + [pltpu.VMEM((SB,), jnp.float32)] * 2
            + [pltpu.VMEM_SHARED((NP,), jnp.float32)]
            + [pltpu.SemaphoreType.DMA] * 4
        ),
    )(body)


_prop1 = _make_prop1(3200)


def _tc_dinv_body(d0, d1, x0, x1, x2, x3, x4, x5, dinv_o, u0, u1, u2, u3, u4, u5):
    dinv = lax.rsqrt(d0[...] + d1[...] + 1.0)
    dinv_o[...] = dinv
    for xr, ur in ((x0, u0), (x1, u1), (x2, u2), (x3, u3), (x4, u4), (x5, u5)):
        ur[...] = dinv * xr[...]


_tc_dinv = pl.pallas_call(
    _tc_dinv_body,
    out_shape=[jax.ShapeDtypeStruct((RN, CN), jnp.float32)] * 7,
)


def _tc_l1_body(*refs):
    dinv_r = refs[0]
    a0 = refs[1:7]
    a1 = refs[7:13]
    u0 = refs[13:19]
    w11, b11, w12 = refs[19], refs[20], refs[21]
    out_u = refs[22]
    dinv = dinv_r[...]
    q = [dinv * (a0[j][...] + a1[j][...] + u0[j][...]) for j in range(6)]
    t = jnp.zeros((RN, CN), jnp.float32)
    for j in range(16):
        h = b11[j]
        for i in range(6):
            h = h + q[i] * w11[i * 16 + j]
        t = t + _lrelu(h) * w12[j]
    out_u[...] = dinv * t


_tc_l1 = pl.pallas_call(
    _tc_l1_body,
    out_shape=jax.ShapeDtypeStruct((RN, CN), jnp.float32),
)


def _tc_ymap_body(dinv_r, a0, a1, u, b, y_o, un_o):
    dinv = dinv_r[...]
    y = _lrelu(dinv * (a0[...] + a1[...] + u[...]) + b[0])
    y_o[...] = y
    un_o[...] = dinv * y


_tc_ymap = pl.pallas_call(
    _tc_ymap_body,
    out_shape=[jax.ShapeDtypeStruct((RN, CN), jnp.float32)] * 2,
)


def _tc_tmap_body(dinv_r, a0, a1, u, wa, ba, wb, un_o):
    dinv = dinv_r[...]
    z = dinv * (a0[...] + a1[...] + u[...])
    t = jnp.zeros((RN, CN), jnp.float32)
    for j in range(16):
        t = t + _lrelu(z * wa[j] + ba[j]) * wb[j]
    un_o[...] = dinv * t


_tc_tmap = pl.pallas_call(
    _tc_tmap_body,
    out_shape=jax.ShapeDtypeStruct((RN, CN), jnp.float32),
)


def kernel(x, edge_index, W11, b11, W11r, b11r, W12, b12, W21, b21, W22, b22,
           W31, b31, W32, b32):
    ei = edge_index.astype(jnp.int32)
    pad = jnp.full((EP - E,), N, jnp.int32)
    src_p = jnp.concatenate([ei[0], pad])
    dst_p = jnp.concatenate([ei[1], pad])
    zeros = jnp.zeros((NP,), jnp.float32)
    ones_kr = jnp.ones((SB,), jnp.float32)
    xp = jnp.pad(x, ((0, NP - N), (0, 0)))

    degp = _deg_sc(dst_p, ones_kr, zeros)
    xcols = [xp[:, j].reshape(RN, CN) for j in range(6)]
    dinv2, *u0 = _tc_dinv(degp[0].reshape(RN, CN), degp[1].reshape(RN, CN),
                          *xcols)

    acc6 = [_prop1(src_p, dst_p, u.reshape(NP), zeros) for u in u0]
    w11f, w12f = W11.reshape(-1), W12.reshape(-1)
    w11rf = W11r.reshape(-1)
    w21f, w22f = W21.reshape(-1), W22.reshape(-1)
    w31f, w32f = W31.reshape(-1), W32.reshape(-1)
    u = _tc_l1(dinv2,
               *[a[0].reshape(RN, CN) for a in acc6],
               *[a[1].reshape(RN, CN) for a in acc6],
               *u0, w11f, b11, w12f)

    seq = [
        (b12, (w11rf, b11r, w12f)),
        (b12, (w21f, b21, w22f)),
        (b22, (w21f, b21, w22f)),
        (b22, (w31f, b31, w32f)),
        (b32, (w31f, b31, w32f)),
        (b32, None),
    ]
    ys = []
    for bout, nxt in seq:
        acc = _prop1(src_p, dst_p, u.reshape(NP), zeros)
        y, uy = _tc_ymap(dinv2, acc[0].reshape(RN, CN), acc[1].reshape(RN, CN),
                         u, bout)
        ys.append(y)
        if nxt is not None:
            wa, ba, wb = nxt
            acc = _prop1(src_p, dst_p, uy.reshape(NP), zeros)
            u = _tc_tmap(dinv2, acc[0].reshape(RN, CN),
                         acc[1].reshape(RN, CN), uy, wa, ba, wb)

    return tuple(y.reshape(NP)[:N].reshape(N, 1) for y in ys)


# speedup vs baseline: 3.7494x; 1.1074x over previous
import functools

import jax
import jax.numpy as jnp
from jax import lax
from jax.experimental import pallas as pl
from jax.experimental.pallas import tpu as pltpu
from jax.experimental.pallas import tpu_sc as plsc

N = 100000
E = 3200000
RN, CN = 784, 128
NP = RN * CN
NC, NS = 2, 16
NW = NC * NS
EPW = 102400
EP = EPW * NW
SB = 6400
SB6 = 3200
NPS = NP // NS

_MESH = plsc.VectorSubcoreMesh(core_axis_name="c", subcore_axis_name="s")


def _lrelu(h):
    return jnp.where(h >= 0, h, 0.1 * h)


def _deg_body(dst_hbm, ones_hbm, zeros_hbm, out_hbm,
              dstb0, dstb1, ones_v, acc_sh, lsem0, lsem1):
    cid = lax.axis_index("c")
    sid = lax.axis_index("s")
    pltpu.sync_copy(zeros_hbm.at[pl.ds(sid * NPS, NPS)],
                    acc_sh.at[pl.ds(sid * NPS, NPS)])
    pltpu.sync_copy(ones_hbm, ones_v)
    plsc.subcore_barrier()
    ebase = (cid * NS + sid) * EPW
    NSC = EPW // SB
    dstb = (dstb0, dstb1)
    lsem = (lsem0, lsem1)

    def lin(s, b, sem):
        return pltpu.async_copy(dst_hbm.at[pl.ds(ebase + s * SB, SB)],
                                dstb[b], sem)

    lin(0, 0, lsem[0])

    def loop(tt, c):
        s0 = 2 * tt
        lin(s0 + 1, 1, lsem[1])
        pltpu.make_async_copy(dst_hbm.at[pl.ds(ebase, SB)],
                              dstb[0], lsem[0]).wait()
        pltpu.sync_copy(ones_v, acc_sh.at[dstb[0]], add=True)
        lin(jnp.minimum(s0 + 2, NSC - 1), 0, lsem[0])
        pltpu.make_async_copy(dst_hbm.at[pl.ds(ebase, SB)],
                              dstb[1], lsem[1]).wait()
        pltpu.sync_copy(ones_v, acc_sh.at[dstb[1]], add=True)
        return c

    lax.fori_loop(0, NSC // 2, loop, 0)
    pltpu.make_async_copy(dst_hbm.at[pl.ds(ebase, SB)], dstb[0], lsem[0]).wait()
    plsc.subcore_barrier()
    pltpu.sync_copy(acc_sh.at[pl.ds(sid * NPS, NPS)],
                    out_hbm.at[cid, pl.ds(sid * NPS, NPS)])


_deg_sc = functools.partial(
    pl.kernel,
    out_type=jax.ShapeDtypeStruct((NC, NP), jnp.float32),
    mesh=_MESH,
    scratch_types=[
        pltpu.VMEM((SB,), jnp.int32),
        pltpu.VMEM((SB,), jnp.int32),
        pltpu.VMEM((SB,), jnp.float32),
        pltpu.VMEM_SHARED((NP,), jnp.float32),
        pltpu.SemaphoreType.DMA,
        pltpu.SemaphoreType.DMA,
    ],
)(_deg_body)


def _make_prop1(SB):
    NSC = EPW // SB

    def body(*refs):
        (src_hbm, dst_hbm, u_hbm, zeros_hbm, out_hbm, u_loc) = refs[:6]
        srcb = refs[6:10]
        dstb = refs[10:14]
        vals = refs[14:18]
        acc_sh = refs[18]
        lsem = refs[19:23]
        ssem = refs[23:27]
        cid = lax.axis_index("c")
        sid = lax.axis_index("s")
        pltpu.sync_copy(zeros_hbm.at[pl.ds(sid * NPS, NPS)],
                        acc_sh.at[pl.ds(sid * NPS, NPS)])
        pltpu.sync_copy(u_hbm, u_loc)
        plsc.subcore_barrier()
        ebase = (cid * NS + sid) * EPW

        def lin(c, r):
            off = ebase + c * SB
            pltpu.async_copy(src_hbm.at[pl.ds(off, SB)], srcb[r], lsem[r])
            pltpu.async_copy(dst_hbm.at[pl.ds(off, SB)], dstb[r], lsem[r])

        def wait_lin(r):
            pltpu.make_async_copy(src_hbm.at[pl.ds(ebase, SB)],
                                  srcb[r], lsem[r]).wait()
            pltpu.make_async_copy(src_hbm.at[pl.ds(ebase, SB)],
                                  dstb[r], lsem[r]).wait()

        def gather_compute(r):
            def g(i, c):
                o = i * 64
                for q in range(4):
                    idx = srcb[r][pl.ds(o + q * 16, 16)]
                    vals[r][pl.ds(o + q * 16, 16)] = plsc.load_gather(
                        u_loc, [idx])
                return c
            lax.fori_loop(0, SB // 64, g, 0)

        def issue_scatter(r):
            pltpu.async_copy(vals[r], acc_sh.at[dstb[r]], ssem[r], add=True)

        def wait_scatter(r):
            pltpu.make_async_copy(vals[r], acc_sh.at[dstb[r]], ssem[r]).wait()

        for r in range(4):
            lin(r, r)
        for r in range(4):
            wait_lin(r)
            gather_compute(r)
            issue_scatter(r)
            lin(4 + r, r)

        def loop(t, c):
            s0 = 4 * t
            for r in range(4):
                wait_lin(r)
                wait_scatter(r)
                gather_compute(r)
                issue_scatter(r)
                lin(jnp.minimum(s0 + 4 + r, NSC - 1), r)
            return c

        lax.fori_loop(1, NSC // 4, loop, 0)
        for r in range(4):
            wait_lin(r)
            wait_scatter(r)
        plsc.subcore_barrier()
        pltpu.sync_copy(acc_sh.at[pl.ds(sid * NPS, NPS)],
                        out_hbm.at[cid, pl.ds(sid * NPS, NPS)])

    return functools.partial(
        pl.kernel,
        out_type=jax.ShapeDtypeStruct((NC, NP), jnp.float32),
        mesh=_MESH,
        compiler_params=pltpu.CompilerParams(needs_layout_passes=False),
        scratch_types=(
            [pltpu.VMEM((NP,), jnp.float32)]
            + [pltpu.VMEM((SB,), jnp.int32)] * 8
            + [pltpu.VMEM((SB,), jnp.float32)] * 4
            + [pltpu.VMEM_SHARED((NP,), jnp.float32)]
            + [pltpu.SemaphoreType.DMA] * 8
        ),
    )(body)


_prop1 = _make_prop1(1600)


def _tc_dinv_body(d0, d1, x0, x1, x2, x3, x4, x5, dinv_o, u0, u1, u2, u3, u4, u5):
    dinv = lax.rsqrt(d0[...] + d1[...] + 1.0)
    dinv_o[...] = dinv
    for xr, ur in ((x0, u0), (x1, u1), (x2, u2), (x3, u3), (x4, u4), (x5, u5)):
        ur[...] = dinv * xr[...]


_tc_dinv = pl.pallas_call(
    _tc_dinv_body,
    out_shape=[jax.ShapeDtypeStruct((RN, CN), jnp.float32)] * 7,
)


def _tc_l1_body(*refs):
    dinv_r = refs[0]
    a0 = refs[1:7]
    a1 = refs[7:13]
    u0 = refs[13:19]
    w11, b11, w12 = refs[19], refs[20], refs[21]
    out_u = refs[22]
    dinv = dinv_r[...]
    q = [dinv * (a0[j][...] + a1[j][...] + u0[j][...]) for j in range(6)]
    t = jnp.zeros((RN, CN), jnp.float32)
    for j in range(16):
        h = b11[j]
        for i in range(6):
            h = h + q[i] * w11[i * 16 + j]
        t = t + _lrelu(h) * w12[j]
    out_u[...] = dinv * t


_tc_l1 = pl.pallas_call(
    _tc_l1_body,
    out_shape=jax.ShapeDtypeStruct((RN, CN), jnp.float32),
)


def _tc_ymap_body(dinv_r, a0, a1, u, b, y_o, un_o):
    dinv = dinv_r[...]
    y = _lrelu(dinv * (a0[...] + a1[...] + u[...]) + b[0])
    y_o[...] = y
    un_o[...] = dinv * y


_tc_ymap = pl.pallas_call(
    _tc_ymap_body,
    out_shape=[jax.ShapeDtypeStruct((RN, CN), jnp.float32)] * 2,
)


def _tc_tmap_body(dinv_r, a0, a1, u, wa, ba, wb, un_o):
    dinv = dinv_r[...]
    z = dinv * (a0[...] + a1[...] + u[...])
    t = jnp.zeros((RN, CN), jnp.float32)
    for j in range(16):
        t = t + _lrelu(z * wa[j] + ba[j]) * wb[j]
    un_o[...] = dinv * t


_tc_tmap = pl.pallas_call(
    _tc_tmap_body,
    out_shape=jax.ShapeDtypeStruct((RN, CN), jnp.float32),
)


def kernel(x, edge_index, W11, b11, W11r, b11r, W12, b12, W21, b21, W22, b22,
           W31, b31, W32, b32):
    ei = edge_index.astype(jnp.int32)
    pad = jnp.full((EP - E,), N, jnp.int32)
    src_p = jnp.concatenate([ei[0], pad])
    dst_p = jnp.concatenate([ei[1], pad])
    zeros = jnp.zeros((NP,), jnp.float32)
    ones_kr = jnp.ones((SB,), jnp.float32)
    xp = jnp.pad(x, ((0, NP - N), (0, 0)))

    degp = _deg_sc(dst_p, ones_kr, zeros)
    xcols = [xp[:, j].reshape(RN, CN) for j in range(6)]
    dinv2, *u0 = _tc_dinv(degp[0].reshape(RN, CN), degp[1].reshape(RN, CN),
                          *xcols)

    acc6 = [_prop1(src_p, dst_p, u.reshape(NP), zeros) for u in u0]
    w11f, w12f = W11.reshape(-1), W12.reshape(-1)
    w11rf = W11r.reshape(-1)
    w21f, w22f = W21.reshape(-1), W22.reshape(-1)
    w31f, w32f = W31.reshape(-1), W32.reshape(-1)
    u = _tc_l1(dinv2,
               *[a[0].reshape(RN, CN) for a in acc6],
               *[a[1].reshape(RN, CN) for a in acc6],
               *u0, w11f, b11, w12f)

    seq = [
        (b12, (w11rf, b11r, w12f)),
        (b12, (w21f, b21, w22f)),
        (b22, (w21f, b21, w22f)),
        (b22, (w31f, b31, w32f)),
        (b32, (w31f, b31, w32f)),
        (b32, None),
    ]
    ys = []
    for bout, nxt in seq:
        acc = _prop1(src_p, dst_p, u.reshape(NP), zeros)
        y, uy = _tc_ymap(dinv2, acc[0].reshape(RN, CN), acc[1].reshape(RN, CN),
                         u, bout)
        ys.append(y)
        if nxt is not None:
            wa, ba, wb = nxt
            acc = _prop1(src_p, dst_p, uy.reshape(NP), zeros)
            u = _tc_tmap(dinv2, acc[0].reshape(RN, CN),
                         acc[1].reshape(RN, CN), uy, wa, ba, wb)

    return tuple(y.reshape(NP)[:N].reshape(N, 1) for y in ys)


# speedup vs baseline: 3.9792x; 1.0613x over previous
import functools

import jax
import jax.numpy as jnp
from jax import lax
from jax.experimental import pallas as pl
from jax.experimental.pallas import tpu as pltpu
from jax.experimental.pallas import tpu_sc as plsc

N = 100000
E = 3200000
RN, CN = 784, 128
NP = RN * CN
NC, NS = 2, 16
NW = NC * NS
EPW = 102400
EP = EPW * NW
SB = 6400
SB6 = 3200
NPS = NP // NS

_MESH = plsc.VectorSubcoreMesh(core_axis_name="c", subcore_axis_name="s")


def _lrelu(h):
    return jnp.where(h >= 0, h, 0.1 * h)


def _deg_body(dst_hbm, ones_hbm, zeros_hbm, out_hbm,
              dstb0, dstb1, ones_v, acc_sh, lsem0, lsem1):
    cid = lax.axis_index("c")
    sid = lax.axis_index("s")
    pltpu.sync_copy(zeros_hbm.at[pl.ds(sid * NPS, NPS)],
                    acc_sh.at[pl.ds(sid * NPS, NPS)])
    pltpu.sync_copy(ones_hbm, ones_v)
    plsc.subcore_barrier()
    ebase = (cid * NS + sid) * EPW
    NSC = EPW // SB
    dstb = (dstb0, dstb1)
    lsem = (lsem0, lsem1)

    def lin(s, b, sem):
        return pltpu.async_copy(dst_hbm.at[pl.ds(ebase + s * SB, SB)],
                                dstb[b], sem)

    lin(0, 0, lsem[0])

    def loop(tt, c):
        s0 = 2 * tt
        lin(s0 + 1, 1, lsem[1])
        pltpu.make_async_copy(dst_hbm.at[pl.ds(ebase, SB)],
                              dstb[0], lsem[0]).wait()
        pltpu.sync_copy(ones_v, acc_sh.at[dstb[0]], add=True)
        lin(jnp.minimum(s0 + 2, NSC - 1), 0, lsem[0])
        pltpu.make_async_copy(dst_hbm.at[pl.ds(ebase, SB)],
                              dstb[1], lsem[1]).wait()
        pltpu.sync_copy(ones_v, acc_sh.at[dstb[1]], add=True)
        return c

    lax.fori_loop(0, NSC // 2, loop, 0)
    pltpu.make_async_copy(dst_hbm.at[pl.ds(ebase, SB)], dstb[0], lsem[0]).wait()
    plsc.subcore_barrier()
    pltpu.sync_copy(acc_sh.at[pl.ds(sid * NPS, NPS)],
                    out_hbm.at[cid, pl.ds(sid * NPS, NPS)])


_deg_sc = functools.partial(
    pl.kernel,
    out_type=jax.ShapeDtypeStruct((NC, NP), jnp.float32),
    mesh=_MESH,
    scratch_types=[
        pltpu.VMEM((SB,), jnp.int32),
        pltpu.VMEM((SB,), jnp.int32),
        pltpu.VMEM((SB,), jnp.float32),
        pltpu.VMEM_SHARED((NP,), jnp.float32),
        pltpu.SemaphoreType.DMA,
        pltpu.SemaphoreType.DMA,
    ],
)(_deg_body)


def _make_prop1(SB):
    NSC = EPW // SB

    def body(*refs):
        (src_hbm, dst_hbm, u_hbm, zeros_hbm, out_hbm, u_loc) = refs[:6]
        srcb = refs[6:10]
        dstb = refs[10:14]
        vals = refs[14:18]
        acc_sh = refs[18]
        lsem = refs[19:23]
        ssem = refs[23:27]
        cid = lax.axis_index("c")
        sid = lax.axis_index("s")
        pltpu.sync_copy(zeros_hbm.at[pl.ds(sid * NPS, NPS)],
                        acc_sh.at[pl.ds(sid * NPS, NPS)])
        pltpu.sync_copy(u_hbm, u_loc)
        plsc.subcore_barrier()
        ebase = (cid * NS + sid) * EPW

        def lin(c, r):
            off = ebase + c * SB
            pltpu.async_copy(src_hbm.at[pl.ds(off, SB)], srcb[r], lsem[r])
            pltpu.async_copy(dst_hbm.at[pl.ds(off, SB)], dstb[r], lsem[r])

        def wait_lin(r):
            pltpu.make_async_copy(src_hbm.at[pl.ds(ebase, SB)],
                                  srcb[r], lsem[r]).wait()
            pltpu.make_async_copy(src_hbm.at[pl.ds(ebase, SB)],
                                  dstb[r], lsem[r]).wait()

        def gather_compute(r):
            def g(i, c):
                o = i * 64
                for q in range(4):
                    idx = srcb[r][pl.ds(o + q * 16, 16)]
                    vals[r][pl.ds(o + q * 16, 16)] = plsc.load_gather(
                        u_loc, [idx])
                return c
            lax.fori_loop(0, SB // 64, g, 0)

        def issue_scatter(r):
            pltpu.async_copy(vals[r], acc_sh.at[dstb[r]], ssem[r], add=True)

        def wait_scatter(r):
            pltpu.make_async_copy(vals[r], acc_sh.at[dstb[r]], ssem[r]).wait()

        def step(r, nxt):
            wait_lin(r)
            gather_compute(r)
            issue_scatter(r)
            if nxt is not None:
                lin(nxt, (r + 2) % 4)

        lin(0, 0)
        lin(1, 1)
        step(0, 2)
        step(1, 3)

        def loop(t, c):
            s0 = 2 + 4 * t
            for q in range(4):
                s_chunk = s0 + q
                r = (2 + q) % 4
                wait_lin(r)
                gather_compute(r)
                issue_scatter(r)
                rp = (r + 2) % 4
                wait_scatter(rp)
                lin(s_chunk + 2, rp)
            return c

        lax.fori_loop(0, (NSC - 4) // 4, loop, 0)
        wait_scatter(0)
        step(2, None)
        wait_scatter(1)
        step(3, None)
        wait_scatter(2)
        wait_scatter(3)
        plsc.subcore_barrier()
        pltpu.sync_copy(acc_sh.at[pl.ds(sid * NPS, NPS)],
                        out_hbm.at[cid, pl.ds(sid * NPS, NPS)])

    return functools.partial(
        pl.kernel,
        out_type=jax.ShapeDtypeStruct((NC, NP), jnp.float32),
        mesh=_MESH,
        compiler_params=pltpu.CompilerParams(needs_layout_passes=False),
        scratch_types=(
            [pltpu.VMEM((NP,), jnp.float32)]
            + [pltpu.VMEM((SB,), jnp.int32)] * 8
            + [pltpu.VMEM((SB,), jnp.float32)] * 4
            + [pltpu.VMEM_SHARED((NP,), jnp.float32)]
            + [pltpu.SemaphoreType.DMA] * 8
        ),
    )(body)


_prop1 = _make_prop1(1600)


def _tc_dinv_body(d0, d1, x0, x1, x2, x3, x4, x5, dinv_o, u0, u1, u2, u3, u4, u5):
    dinv = lax.rsqrt(d0[...] + d1[...] + 1.0)
    dinv_o[...] = dinv
    for xr, ur in ((x0, u0), (x1, u1), (x2, u2), (x3, u3), (x4, u4), (x5, u5)):
        ur[...] = dinv * xr[...]


_tc_dinv = pl.pallas_call(
    _tc_dinv_body,
    out_shape=[jax.ShapeDtypeStruct((RN, CN), jnp.float32)] * 7,
)


def _tc_l1_body(*refs):
    dinv_r = refs[0]
    a0 = refs[1:7]
    a1 = refs[7:13]
    u0 = refs[13:19]
    w11, b11, w12 = refs[19], refs[20], refs[21]
    out_u = refs[22]
    dinv = dinv_r[...]
    q = [dinv * (a0[j][...] + a1[j][...] + u0[j][...]) for j in range(6)]
    t = jnp.zeros((RN, CN), jnp.float32)
    for j in range(16):
        h = b11[j]
        for i in range(6):
            h = h + q[i] * w11[i * 16 + j]
        t = t + _lrelu(h) * w12[j]
    out_u[...] = dinv * t


_tc_l1 = pl.pallas_call(
    _tc_l1_body,
    out_shape=jax.ShapeDtypeStruct((RN, CN), jnp.float32),
)


def _tc_ymap_body(dinv_r, a0, a1, u, b, y_o, un_o):
    dinv = dinv_r[...]
    y = _lrelu(dinv * (a0[...] + a1[...] + u[...]) + b[0])
    y_o[...] = y
    un_o[...] = dinv * y


_tc_ymap = pl.pallas_call(
    _tc_ymap_body,
    out_shape=[jax.ShapeDtypeStruct((RN, CN), jnp.float32)] * 2,
)


def _tc_tmap_body(dinv_r, a0, a1, u, wa, ba, wb, un_o):
    dinv = dinv_r[...]
    z = dinv * (a0[...] + a1[...] + u[...])
    t = jnp.zeros((RN, CN), jnp.float32)
    for j in range(16):
        t = t + _lrelu(z * wa[j] + ba[j]) * wb[j]
    un_o[...] = dinv * t


_tc_tmap = pl.pallas_call(
    _tc_tmap_body,
    out_shape=jax.ShapeDtypeStruct((RN, CN), jnp.float32),
)


def kernel(x, edge_index, W11, b11, W11r, b11r, W12, b12, W21, b21, W22, b22,
           W31, b31, W32, b32):
    ei = edge_index.astype(jnp.int32)
    pad = jnp.full((EP - E,), N, jnp.int32)
    src_p = jnp.concatenate([ei[0], pad])
    dst_p = jnp.concatenate([ei[1], pad])
    zeros = jnp.zeros((NP,), jnp.float32)
    ones_kr = jnp.ones((SB,), jnp.float32)
    xp = jnp.pad(x, ((0, NP - N), (0, 0)))

    degp = _deg_sc(dst_p, ones_kr, zeros)
    xcols = [xp[:, j].reshape(RN, CN) for j in range(6)]
    dinv2, *u0 = _tc_dinv(degp[0].reshape(RN, CN), degp[1].reshape(RN, CN),
                          *xcols)

    acc6 = [_prop1(src_p, dst_p, u.reshape(NP), zeros) for u in u0]
    w11f, w12f = W11.reshape(-1), W12.reshape(-1)
    w11rf = W11r.reshape(-1)
    w21f, w22f = W21.reshape(-1), W22.reshape(-1)
    w31f, w32f = W31.reshape(-1), W32.reshape(-1)
    u = _tc_l1(dinv2,
               *[a[0].reshape(RN, CN) for a in acc6],
               *[a[1].reshape(RN, CN) for a in acc6],
               *u0, w11f, b11, w12f)

    seq = [
        (b12, (w11rf, b11r, w12f)),
        (b12, (w21f, b21, w22f)),
        (b22, (w21f, b21, w22f)),
        (b22, (w31f, b31, w32f)),
        (b32, (w31f, b31, w32f)),
        (b32, None),
    ]
    ys = []
    for bout, nxt in seq:
        acc = _prop1(src_p, dst_p, u.reshape(NP), zeros)
        y, uy = _tc_ymap(dinv2, acc[0].reshape(RN, CN), acc[1].reshape(RN, CN),
                         u, bout)
        ys.append(y)
        if nxt is not None:
            wa, ba, wb = nxt
            acc = _prop1(src_p, dst_p, uy.reshape(NP), zeros)
            u = _tc_tmap(dinv2, acc[0].reshape(RN, CN),
                         acc[1].reshape(RN, CN), uy, wa, ba, wb)

    return tuple(y.reshape(NP)[:N].reshape(N, 1) for y in ys)
